# initial kernel scaffold (unmeasured)
import functools

import jax
import jax.numpy as jnp
from jax import lax
from jax.experimental import pallas as pl
from jax.experimental.pallas import tpu as pltpu

N_DEV = 8
B, Sq, Hq, Dh = 2, 128, 4, 64
SKV_PER = 128
SKV = N_DEV * SKV_PER
DM = 512
DQK = Hq * Dh


def kernel(x, Wq, K_ext, V_ext, Wo):
    def body(x_ref, wq_ref, k_ref, v_ref, wo_ref, out_ref,
             k_all, v_all, ksend_sems, krecv_sems, vsend_sems, vrecv_sems):
        my = lax.axis_index("i")
        left = lax.rem(my - 1 + N_DEV, N_DEV)
        right = lax.rem(my + 1, N_DEV)

        barrier_sem = pltpu.get_barrier_semaphore()
        for nbr in (left, right):
            pl.semaphore_signal(
                barrier_sem, inc=1,
                device_id=(nbr,), device_id_type=pl.DeviceIdType.MESH,
            )
        pl.semaphore_wait(barrier_sem, 2)

        k_all[my] = k_ref[...].astype(jnp.bfloat16)
        v_all[my] = v_ref[...].astype(jnp.bfloat16)

        for h in range(N_DEV - 1):
            send_o = lax.rem(my - h + N_DEV, N_DEV)
            recv_o = lax.rem(my - h - 1 + N_DEV, N_DEV)
            k_rdma = pltpu.make_async_remote_copy(
                src_ref=k_all.at[send_o],
                dst_ref=k_all.at[send_o],
                send_sem=ksend_sems.at[h],
                recv_sem=krecv_sems.at[recv_o],
                device_id=(right,),
                device_id_type=pl.DeviceIdType.MESH,
            )
            v_rdma = pltpu.make_async_remote_copy(
                src_ref=v_all.at[send_o],
                dst_ref=v_all.at[send_o],
                send_sem=vsend_sems.at[h],
                recv_sem=vrecv_sems.at[recv_o],
                device_id=(right,),
                device_id_type=pl.DeviceIdType.MESH,
            )
            k_rdma.start()
            v_rdma.start()
            k_rdma.wait()
            v_rdma.wait()

        qi = lax.broadcasted_iota(jnp.int32, (Sq, SKV), 0)
        ki = lax.broadcasted_iota(jnp.int32, (Sq, SKV), 1)
        mask = (jnp.abs(qi - ki) <= 128) | (ki < 32) | (qi < 32)

        wq_bf = wq_ref[...].astype(jnp.bfloat16)
        wo_bf = wo_ref[...].astype(jnp.bfloat16)

        for b in range(B):
            xb = x_ref[b].astype(jnp.bfloat16)
            q_b = jnp.dot(xb, wq_bf,
                          preferred_element_type=jnp.float32)
            q_b = q_b.astype(jnp.bfloat16)
            ctxs = []
            for h in range(Hq):
                q_bh = q_b[:, h * Dh:(h + 1) * Dh]
                k_full = jnp.concatenate(
                    [k_all[o, b, :, h, :] for o in range(N_DEV)], axis=0
                )
                v_full = jnp.concatenate(
                    [v_all[o, b, :, h, :] for o in range(N_DEV)], axis=0
                )
                s = lax.dot_general(
                    q_bh, k_full,
                    (((1,), (1,)), ((), ())),
                    preferred_element_type=jnp.float32,
                ) * 0.125
                s = jnp.where(mask, s, -1e9)
                m = jnp.max(s, axis=1, keepdims=True)
                w = jnp.exp(s - m)
                w = w / jnp.sum(w, axis=1, keepdims=True)
                ctx = jnp.dot(w.astype(jnp.bfloat16), v_full,
                              preferred_element_type=jnp.float32)
                ctxs.append(ctx.astype(jnp.bfloat16))
            ctx_b = jnp.concatenate(ctxs, axis=1)
            out_ref[b] = jnp.dot(ctx_b, wo_bf,
                                 preferred_element_type=jnp.float32)

    return pl.pallas_call(
        body,
        out_shape=jax.ShapeDtypeStruct((B, Sq, DM), jnp.float32),
        in_specs=[pl.BlockSpec(memory_space=pltpu.VMEM)] * 5,
        out_specs=pl.BlockSpec(memory_space=pltpu.VMEM),
        scratch_shapes=[
            pltpu.VMEM((N_DEV, B, SKV_PER, Hq, Dh), jnp.bfloat16),
            pltpu.VMEM((N_DEV, B, SKV_PER, Hq, Dh), jnp.bfloat16),
            pltpu.SemaphoreType.DMA((N_DEV,)),
            pltpu.SemaphoreType.DMA((N_DEV,)),
            pltpu.SemaphoreType.DMA((N_DEV,)),
            pltpu.SemaphoreType.DMA((N_DEV,)),
        ],
        compiler_params=pltpu.CompilerParams(collective_id=0),
    )(x, Wq, K_ext, V_ext, Wo)


# baseline (device time: 71469 ns/iter reference)
import jax
import jax.numpy as jnp
from jax import lax
from jax.experimental import pallas as pl
from jax.experimental.pallas import tpu as pltpu

N_DEV = 8
B, Sq, Hq, Dh = 2, 128, 4, 64
SKV_PER = 128
SKV = N_DEV * SKV_PER
DM = 512
DQK = Hq * Dh


def kernel(x, Wq, K_ext, V_ext, Wo):
    def body(x_ref, wq_ref, k_ref, v_ref, wo_ref, out_ref,
             k_all, v_all, ksend_sems, krecv_sems, vsend_sems, vrecv_sems):
        my = lax.axis_index("i")
        left = lax.rem(my - 1 + N_DEV, N_DEV)
        right = lax.rem(my + 1, N_DEV)

        barrier_sem = pltpu.get_barrier_semaphore()
        for nbr in (left, right):
            pl.semaphore_signal(
                barrier_sem, inc=1,
                device_id=(nbr,), device_id_type=pl.DeviceIdType.MESH,
            )
        pl.semaphore_wait(barrier_sem, 2)

        k_all[0] = k_ref[...].astype(jnp.bfloat16)
        v_all[0] = v_ref[...].astype(jnp.bfloat16)

        for h in range(N_DEV - 1):
            k_rdma = pltpu.make_async_remote_copy(
                src_ref=k_all.at[h],
                dst_ref=k_all.at[h + 1],
                send_sem=ksend_sems.at[h],
                recv_sem=krecv_sems.at[h + 1],
                device_id=(right,),
                device_id_type=pl.DeviceIdType.MESH,
            )
            v_rdma = pltpu.make_async_remote_copy(
                src_ref=v_all.at[h],
                dst_ref=v_all.at[h + 1],
                send_sem=vsend_sems.at[h],
                recv_sem=vrecv_sems.at[h + 1],
                device_id=(right,),
                device_id_type=pl.DeviceIdType.MESH,
            )
            k_rdma.start()
            v_rdma.start()
            k_rdma.wait()
            v_rdma.wait()

        qi = lax.broadcasted_iota(jnp.int32, (Sq, SKV), 0)
        col = lax.broadcasted_iota(jnp.int32, (Sq, SKV), 1)
        slot = col // SKV_PER
        ki = jnp.remainder(my - slot + N_DEV, N_DEV) * SKV_PER + (
            col % SKV_PER)
        mask = (jnp.abs(qi - ki) <= 128) | (ki < 32) | (qi < 32)

        wq_bf = wq_ref[...].astype(jnp.bfloat16)
        wo_bf = wo_ref[...].astype(jnp.bfloat16)

        for b in range(B):
            xb = x_ref[b].astype(jnp.bfloat16)
            q_b = jnp.dot(xb, wq_bf,
                          preferred_element_type=jnp.float32)
            q_b = q_b.astype(jnp.bfloat16)
            ctxs = []
            for hh in range(Hq):
                q_bh = q_b[:, hh * Dh:(hh + 1) * Dh]
                k_full = jnp.concatenate(
                    [k_all[j, b, :, hh, :] for j in range(N_DEV)], axis=0
                )
                v_full = jnp.concatenate(
                    [v_all[j, b, :, hh, :] for j in range(N_DEV)], axis=0
                )
                s = lax.dot_general(
                    q_bh, k_full,
                    (((1,), (1,)), ((), ())),
                    preferred_element_type=jnp.float32,
                ) * 0.125
                s = jnp.where(mask, s, -1e9)
                m = jnp.max(s, axis=1, keepdims=True)
                w = jnp.exp(s - m)
                w = w / jnp.sum(w, axis=1, keepdims=True)
                ctx = jnp.dot(w.astype(jnp.bfloat16), v_full,
                              preferred_element_type=jnp.float32)
                ctxs.append(ctx.astype(jnp.bfloat16))
            ctx_b = jnp.concatenate(ctxs, axis=1)
            out_ref[b] = jnp.dot(ctx_b, wo_bf,
                                 preferred_element_type=jnp.float32)

    return pl.pallas_call(
        body,
        out_shape=jax.ShapeDtypeStruct((B, Sq, DM), jnp.float32),
        in_specs=[pl.BlockSpec(memory_space=pltpu.VMEM)] * 5,
        out_specs=pl.BlockSpec(memory_space=pltpu.VMEM),
        scratch_shapes=[
            pltpu.VMEM((N_DEV, B, SKV_PER, Hq, Dh), jnp.bfloat16),
            pltpu.VMEM((N_DEV, B, SKV_PER, Hq, Dh), jnp.bfloat16),
            pltpu.SemaphoreType.DMA((N_DEV,)),
            pltpu.SemaphoreType.DMA((N_DEV,)),
            pltpu.SemaphoreType.DMA((N_DEV,)),
            pltpu.SemaphoreType.DMA((N_DEV,)),
        ],
        compiler_params=pltpu.CompilerParams(collective_id=0),
    )(x, Wq, K_ext, V_ext, Wo)


# device time: 28758 ns/iter; 2.4852x vs baseline; 2.4852x over previous
import jax
import jax.numpy as jnp
from jax import lax
from jax.experimental import pallas as pl
from jax.experimental.pallas import tpu as pltpu

N_DEV = 8
B, Sq, Hq, Dh = 2, 128, 4, 64
SKV_PER = 128
DM = 512
DQK = Hq * Dh


def kernel(x, Wq, K_ext, V_ext, Wo):
    def body(x_ref, wq_ref, k_ref, v_ref, wo_ref, out_ref,
             o_all, st_all, osend_sems, orecv_sems, ssend_sems, srecv_sems):
        my = lax.axis_index("i")

        barrier_sem = pltpu.get_barrier_semaphore()
        for d in range(1, N_DEV):
            peer = lax.rem(my + d, N_DEV)
            pl.semaphore_signal(
                barrier_sem, inc=1,
                device_id=(peer,), device_id_type=pl.DeviceIdType.MESH,
            )
        pl.semaphore_wait(barrier_sem, N_DEV - 1)

        qi = lax.broadcasted_iota(jnp.int32, (Sq, SKV_PER), 0)
        cc = lax.broadcasted_iota(jnp.int32, (Sq, SKV_PER), 1)
        ki = my * SKV_PER + cc
        mask = (jnp.abs(qi - ki) <= 128) | (ki < 32) | (qi < 32)

        wq_bf = wq_ref[...].astype(jnp.bfloat16)
        for b in range(B):
            xb = x_ref[b].astype(jnp.bfloat16)
            q_b = jnp.dot(xb, wq_bf,
                          preferred_element_type=jnp.float32)
            q_b = q_b.astype(jnp.bfloat16)
            for hh in range(Hq):
                q_bh = q_b[:, hh * Dh:(hh + 1) * Dh]
                k_loc = k_ref[b, :, hh, :].astype(jnp.bfloat16)
                v_loc = v_ref[b, :, hh, :].astype(jnp.bfloat16)
                s = lax.dot_general(
                    q_bh, k_loc,
                    (((1,), (1,)), ((), ())),
                    preferred_element_type=jnp.float32,
                ) * 0.125
                s = jnp.where(mask, s, -1e9)
                m = jnp.max(s, axis=1)
                w = jnp.exp(s - m[:, None])
                l = jnp.sum(w, axis=1)
                o = jnp.dot(w.astype(jnp.bfloat16), v_loc,
                            preferred_element_type=jnp.float32)
                o_all[0, b, hh] = o.astype(jnp.bfloat16)
                st_all[0, 0, b, hh] = m
                st_all[0, 1, b, hh] = l

        rdmas = []
        for d in range(1, N_DEV):
            peer = lax.rem(my + d, N_DEV)
            o_rdma = pltpu.make_async_remote_copy(
                src_ref=o_all.at[0],
                dst_ref=o_all.at[d],
                send_sem=osend_sems.at[d],
                recv_sem=orecv_sems.at[d],
                device_id=(peer,),
                device_id_type=pl.DeviceIdType.MESH,
            )
            s_rdma = pltpu.make_async_remote_copy(
                src_ref=st_all.at[0],
                dst_ref=st_all.at[d],
                send_sem=ssend_sems.at[d],
                recv_sem=srecv_sems.at[d],
                device_id=(peer,),
                device_id_type=pl.DeviceIdType.MESH,
            )
            o_rdma.start()
            s_rdma.start()
            rdmas.append((o_rdma, s_rdma))
        for o_rdma, s_rdma in rdmas:
            o_rdma.wait_recv()
            s_rdma.wait_recv()

        wo_bf = wo_ref[...].astype(jnp.bfloat16)
        for b in range(B):
            m_stack = jnp.stack(
                [st_all[j, 0, b] for j in range(N_DEV)])
            l_stack = jnp.stack(
                [st_all[j, 1, b] for j in range(N_DEV)])
            m_tot = jnp.max(m_stack, axis=0)
            scale = jnp.exp(m_stack - m_tot[None])
            l_tot = jnp.sum(scale * l_stack, axis=0)
            ctxs = []
            for hh in range(Hq):
                acc = jnp.zeros((Sq, Dh), jnp.float32)
                for j in range(N_DEV):
                    acc = acc + scale[j, hh][:, None] * (
                        o_all[j, b, hh].astype(jnp.float32))
                ctx = acc / l_tot[hh][:, None]
                ctxs.append(ctx.astype(jnp.bfloat16))
            ctx_b = jnp.concatenate(ctxs, axis=1)
            out_ref[b] = jnp.dot(ctx_b, wo_bf,
                                 preferred_element_type=jnp.float32)

        for o_rdma, s_rdma in rdmas:
            o_rdma.wait_send()
            s_rdma.wait_send()

    return pl.pallas_call(
        body,
        out_shape=jax.ShapeDtypeStruct((B, Sq, DM), jnp.float32),
        in_specs=[pl.BlockSpec(memory_space=pltpu.VMEM)] * 5,
        out_specs=pl.BlockSpec(memory_space=pltpu.VMEM),
        scratch_shapes=[
            pltpu.VMEM((N_DEV, B, Hq, Sq, Dh), jnp.bfloat16),
            pltpu.VMEM((N_DEV, 2, B, Hq, Sq), jnp.float32),
            pltpu.SemaphoreType.DMA((N_DEV,)),
            pltpu.SemaphoreType.DMA((N_DEV,)),
            pltpu.SemaphoreType.DMA((N_DEV,)),
            pltpu.SemaphoreType.DMA((N_DEV,)),
        ],
        compiler_params=pltpu.CompilerParams(collective_id=0),
    )(x, Wq, K_ext, V_ext, Wo)


# device time: 22597 ns/iter; 3.1628x vs baseline; 1.2726x over previous
import jax
import jax.numpy as jnp
from jax import lax
from jax.experimental import pallas as pl
from jax.experimental.pallas import tpu as pltpu

N_DEV = 8
B, Sq, Hq, Dh = 2, 128, 4, 64
SKV_PER = 128
DM = 512
DQK = Hq * Dh


def kernel(x, Wq, K_ext, V_ext, Wo):
    def body(x_ref, wq_ref, k_ref, v_ref, wo_ref, out_ref,
             o_all, st_all, osend_sems, orecv_sems, ssend_sems, srecv_sems):
        my = lax.axis_index("i")

        barrier_sem = pltpu.get_barrier_semaphore()
        for d in range(1, N_DEV):
            peer = lax.rem(my + d, N_DEV)
            pl.semaphore_signal(
                barrier_sem, inc=1,
                device_id=(peer,), device_id_type=pl.DeviceIdType.MESH,
            )

        qi = lax.broadcasted_iota(jnp.int32, (Sq, SKV_PER), 0)
        cc = lax.broadcasted_iota(jnp.int32, (Sq, SKV_PER), 1)
        ki = my * SKV_PER + cc
        mask = (jnp.abs(qi - ki) <= 128) | (ki < 32) | (qi < 32)

        wq_bf = wq_ref[...].astype(jnp.bfloat16)
        barrier_done = False
        rdmas = []
        for b in range(B):
            xb = x_ref[b].astype(jnp.bfloat16)
            q_b = jnp.dot(xb, wq_bf,
                          preferred_element_type=jnp.float32)
            q_b = q_b.astype(jnp.bfloat16)
            for hh in range(Hq):
                q_bh = q_b[:, hh * Dh:(hh + 1) * Dh]
                k_loc = k_ref[b, :, hh, :].astype(jnp.bfloat16)
                v_loc = v_ref[b, :, hh, :].astype(jnp.bfloat16)
                s = lax.dot_general(
                    q_bh, k_loc,
                    (((1,), (1,)), ((), ())),
                    preferred_element_type=jnp.float32,
                ) * 0.125
                s = jnp.where(mask, s, -1e9)
                m = jnp.max(s, axis=1)
                w = jnp.exp(s - m[:, None])
                l = jnp.sum(w, axis=1)
                o = jnp.dot(w.astype(jnp.bfloat16), v_loc,
                            preferred_element_type=jnp.float32)
                o_all[0, b, hh] = o.astype(jnp.bfloat16)
                st_all[0, b, 0, hh] = m
                st_all[0, b, 1, hh] = l

            if not barrier_done:
                pl.semaphore_wait(barrier_sem, N_DEV - 1)
                barrier_done = True
            for d in range(1, N_DEV):
                peer = lax.rem(my + d, N_DEV)
                s_rdma = pltpu.make_async_remote_copy(
                    src_ref=st_all.at[0, b],
                    dst_ref=st_all.at[d, b],
                    send_sem=ssend_sems.at[d, b],
                    recv_sem=srecv_sems.at[d, b],
                    device_id=(peer,),
                    device_id_type=pl.DeviceIdType.MESH,
                )
                o_rdma = pltpu.make_async_remote_copy(
                    src_ref=o_all.at[0, b],
                    dst_ref=o_all.at[d, b],
                    send_sem=osend_sems.at[d, b],
                    recv_sem=orecv_sems.at[d, b],
                    device_id=(peer,),
                    device_id_type=pl.DeviceIdType.MESH,
                )
                s_rdma.start()
                o_rdma.start()
                rdmas.append((o_rdma, s_rdma))

        wo_bf = wo_ref[...].astype(jnp.bfloat16)
        for b in range(B):
            for d in range(1, N_DEV):
                rdmas[b * (N_DEV - 1) + d - 1][1].wait_recv()
            m_stack = jnp.stack(
                [st_all[j, b, 0] for j in range(N_DEV)])
            l_stack = jnp.stack(
                [st_all[j, b, 1] for j in range(N_DEV)])
            m_tot = jnp.max(m_stack, axis=0)
            scale = jnp.exp(m_stack - m_tot[None])
            l_tot = jnp.sum(scale * l_stack, axis=0)
            accs = [scale[0, hh][:, None] * o_all[0, b, hh].astype(jnp.float32)
                    for hh in range(Hq)]
            for d in range(1, N_DEV):
                rdmas[b * (N_DEV - 1) + d - 1][0].wait_recv()
                for hh in range(Hq):
                    accs[hh] = accs[hh] + scale[d, hh][:, None] * (
                        o_all[d, b, hh].astype(jnp.float32))
            ctxs = [(accs[hh] / l_tot[hh][:, None]).astype(jnp.bfloat16)
                    for hh in range(Hq)]
            ctx_b = jnp.concatenate(ctxs, axis=1)
            out_ref[b] = jnp.dot(ctx_b, wo_bf,
                                 preferred_element_type=jnp.float32)

        for o_rdma, s_rdma in rdmas:
            o_rdma.wait_send()
            s_rdma.wait_send()

    return pl.pallas_call(
        body,
        out_shape=jax.ShapeDtypeStruct((B, Sq, DM), jnp.float32),
        in_specs=[pl.BlockSpec(memory_space=pltpu.VMEM)] * 5,
        out_specs=pl.BlockSpec(memory_space=pltpu.VMEM),
        scratch_shapes=[
            pltpu.VMEM((N_DEV, B, Hq, Sq, Dh), jnp.bfloat16),
            pltpu.VMEM((N_DEV, B, 2, Hq, Sq), jnp.float32),
            pltpu.SemaphoreType.DMA((N_DEV, B)),
            pltpu.SemaphoreType.DMA((N_DEV, B)),
            pltpu.SemaphoreType.DMA((N_DEV, B)),
            pltpu.SemaphoreType.DMA((N_DEV, B)),
        ],
        compiler_params=pltpu.CompilerParams(collective_id=0),
    )(x, Wq, K_ext, V_ext, Wo)


# device time: 15978 ns/iter; 4.4730x vs baseline; 1.4143x over previous
import jax
import jax.numpy as jnp
from jax import lax
from jax.experimental import pallas as pl
from jax.experimental.pallas import tpu as pltpu

N_DEV = 8
B, Sq, Hq, Dh = 2, 128, 4, 64
SKV_PER = 128
DM = 512
DQK = Hq * Dh
SQ_GLOB = 32


def kernel(x, Wq, K_ext, V_ext, Wo):
    def body(x_ref, wq_ref, k_ref, v_ref, wo_ref, out_ref,
             o_all, st_all, osend_sems, orecv_sems, ssend_sems, srecv_sems):
        my = lax.axis_index("i")
        i_am_full = my < 2

        for d in range(1, N_DEV):
            for b in range(B):
                o_all[d, b, SQ_GLOB:, :] = jnp.zeros(
                    (Sq - SQ_GLOB, DQK), jnp.bfloat16)

        barrier_sem = pltpu.get_barrier_semaphore()
        for d in range(1, N_DEV):
            peer = lax.rem(my + d, N_DEV)
            pl.semaphore_signal(
                barrier_sem, inc=1,
                device_id=(peer,), device_id_type=pl.DeviceIdType.MESH,
            )

        qi = lax.broadcasted_iota(jnp.int32, (Sq, SKV_PER), 0)
        cc = lax.broadcasted_iota(jnp.int32, (Sq, SKV_PER), 1)
        ki = my * SKV_PER + cc
        mask = (jnp.abs(qi - ki) <= 128) | (ki < 32) | (qi < 32)

        x_all = x_ref[...].reshape(B * Sq, DM)
        q_all = jnp.dot(x_all, wq_ref[...],
                        preferred_element_type=jnp.float32)
        q_all = (q_all * 0.125).astype(jnp.bfloat16)

        barrier_done = False
        rdmas = []
        for b in range(B):
            k_pack = k_ref[b]
            v_pack = v_ref[b]
            for hh in range(Hq):
                q_bh = q_all[b * Sq:(b + 1) * Sq, hh * Dh:(hh + 1) * Dh]
                k_loc = k_pack[:, hh * Dh:(hh + 1) * Dh]
                v_loc = v_pack[:, hh * Dh:(hh + 1) * Dh]
                s = lax.dot_general(
                    q_bh, k_loc,
                    (((1,), (1,)), ((), ())),
                    preferred_element_type=jnp.float32,
                )
                w = jnp.exp(jnp.where(mask, s, -1e9))
                l = jnp.sum(w, axis=1)
                o = jnp.dot(w.astype(jnp.bfloat16), v_loc,
                            preferred_element_type=jnp.float32)
                o_all[0, b, :, hh * Dh:(hh + 1) * Dh] = o.astype(jnp.bfloat16)
                st_all[0, b, hh] = l

            if not barrier_done:
                pl.semaphore_wait(barrier_sem, N_DEV - 1)
                barrier_done = True
            for d in range(1, N_DEV):
                peer = lax.rem(my + d, N_DEV)
                s_rdma = pltpu.make_async_remote_copy(
                    src_ref=st_all.at[0, b],
                    dst_ref=st_all.at[d, b],
                    send_sem=ssend_sems.at[d, b],
                    recv_sem=srecv_sems.at[d, b],
                    device_id=(peer,),
                    device_id_type=pl.DeviceIdType.MESH,
                )
                s_rdma.start()
                o_full = pltpu.make_async_remote_copy(
                    src_ref=o_all.at[0, b],
                    dst_ref=o_all.at[d, b],
                    send_sem=osend_sems.at[d, b],
                    recv_sem=orecv_sems.at[d, b],
                    device_id=(peer,),
                    device_id_type=pl.DeviceIdType.MESH,
                )
                o_small = pltpu.make_async_remote_copy(
                    src_ref=o_all.at[0, b, pl.ds(0, SQ_GLOB)],
                    dst_ref=o_all.at[d, b, pl.ds(0, SQ_GLOB)],
                    send_sem=osend_sems.at[d, b],
                    recv_sem=orecv_sems.at[d, b],
                    device_id=(peer,),
                    device_id_type=pl.DeviceIdType.MESH,
                )

                @pl.when(i_am_full)
                def _():
                    o_full.start()

                @pl.when(jnp.logical_not(i_am_full))
                def _():
                    o_small.start()

                rdmas.append((o_full, o_small, s_rdma))

        e_row = lax.broadcasted_iota(jnp.int32, (Hq, DQK), 0)
        e_col = lax.broadcasted_iota(jnp.int32, (Hq, DQK), 1)
        e_blk = (e_col // Dh == e_row).astype(jnp.float32)

        def rep(a):
            return lax.dot_general(
                a, e_blk, (((0,), (0,)), ((), ())),
                preferred_element_type=jnp.float32)

        ctxs = []
        for b in range(B):
            for d in range(1, N_DEV):
                rdmas[b * (N_DEV - 1) + d - 1][2].wait_recv()
            l_tot = st_all[0, b][...]
            for j in range(1, N_DEV):
                l_tot = l_tot + st_all[j, b]
            acc = o_all[0, b].astype(jnp.float32)
            for d in range(1, N_DEV):
                o_full, o_small, _ = rdmas[b * (N_DEV - 1) + d - 1]
                sender_full = lax.rem(my - d + N_DEV, N_DEV) < 2

                @pl.when(sender_full)
                def _():
                    o_full.wait_recv()

                @pl.when(jnp.logical_not(sender_full))
                def _():
                    o_small.wait_recv()

                acc = acc + o_all[d, b].astype(jnp.float32)
            ctxs.append((acc * rep(1.0 / l_tot)).astype(jnp.bfloat16))

        ctx_all = jnp.concatenate(ctxs, axis=0)
        out = jnp.dot(ctx_all, wo_ref[...],
                      preferred_element_type=jnp.float32)
        out_ref[...] = out.reshape(B, Sq, DM)

        for o_full, o_small, s_rdma in rdmas:
            s_rdma.wait_send()

            @pl.when(i_am_full)
            def _():
                o_full.wait_send()

            @pl.when(jnp.logical_not(i_am_full))
            def _():
                o_small.wait_send()

    return pl.pallas_call(
        body,
        out_shape=jax.ShapeDtypeStruct((B, Sq, DM), jnp.float32),
        in_specs=[pl.BlockSpec(memory_space=pltpu.VMEM)] * 5,
        out_specs=pl.BlockSpec(memory_space=pltpu.VMEM),
        scratch_shapes=[
            pltpu.VMEM((N_DEV, B, Sq, DQK), jnp.bfloat16),
            pltpu.VMEM((N_DEV, B, Hq, Sq), jnp.float32),
            pltpu.SemaphoreType.DMA((N_DEV, B)),
            pltpu.SemaphoreType.DMA((N_DEV, B)),
            pltpu.SemaphoreType.DMA((N_DEV, B)),
            pltpu.SemaphoreType.DMA((N_DEV, B)),
        ],
        compiler_params=pltpu.CompilerParams(collective_id=0),
    )(x.astype(jnp.bfloat16), Wq.astype(jnp.bfloat16),
      K_ext.reshape(B, SKV_PER, DQK).astype(jnp.bfloat16),
      V_ext.reshape(B, SKV_PER, DQK).astype(jnp.bfloat16),
      Wo.astype(jnp.bfloat16))
